# baseline (speedup 1.0000x reference)
import jax
import jax.numpy as jnp
from jax.experimental import pallas as pl
from jax.experimental.pallas import tpu as pltpu

_THRESH = 0.6
_NMS_T = 0.7
_CASE = 416.0
_K = 512
_NEG = -1e9
_M = 3 * _K
_NR = 96
_NF = 16


def _extract_body(x13_ref, x26_ref, x52_ref, s13_ref, s26_ref, s52_ref):
    for x_ref, s_ref in ((x13_ref, s13_ref), (x26_ref, s26_ref),
                         (x52_ref, s52_ref)):
        s_ref[0] = x_ref[0, :, 0, :]


def _extract_obj(o13, o26, o52):
    N = o13.shape[0]
    hws = tuple(o.shape[3] for o in (o13, o26, o52))
    return pl.pallas_call(
        _extract_body,
        grid=(N,),
        in_specs=[pl.BlockSpec((1, 3, 8, hw), lambda n: (n, 0, 0, 0))
                  for hw in hws],
        out_specs=[pl.BlockSpec((1, 3, hw), lambda n: (n, 0, 0))
                   for hw in hws],
        out_shape=[jax.ShapeDtypeStruct((N, 3, hw), jnp.float32)
                   for hw in hws],
        compiler_params=pltpu.CompilerParams(
            dimension_semantics=("parallel",),
        ),
    )(o13, o26, o52)


def _nms_body(b_ref, bt_ref, out_ref, mf_ref, asm_ref):
    M = _M
    s_c = b_ref[:, 9:10]
    x1c = b_ref[:, 1:2] - 0.5 * b_ref[:, 3:4]
    x2c = b_ref[:, 1:2] + 0.5 * b_ref[:, 3:4]
    y1c = b_ref[:, 2:3] - 0.5 * b_ref[:, 4:5]
    y2c = b_ref[:, 2:3] + 0.5 * b_ref[:, 4:5]
    areac = jnp.maximum(x2c - x1c, 0.0) * jnp.maximum(y2c - y1c, 0.0)
    idxc = jax.lax.broadcasted_iota(jnp.int32, (M, 1), 0)
    s_r = bt_ref[9:10, :]
    x1r = bt_ref[1:2, :] - 0.5 * bt_ref[3:4, :]
    x2r = bt_ref[1:2, :] + 0.5 * bt_ref[3:4, :]
    y1r = bt_ref[2:3, :] - 0.5 * bt_ref[4:5, :]
    y2r = bt_ref[2:3, :] + 0.5 * bt_ref[4:5, :]
    arear = jnp.maximum(x2r - x1r, 0.0) * jnp.maximum(y2r - y1r, 0.0)

    cl = bt_ref[16:96, :]
    mx = jnp.max(cl, axis=0, keepdims=True)
    ii = jax.lax.broadcasted_iota(jnp.int32, (80, M), 0)
    cls_r = jnp.min(jnp.where(cl == mx, ii, 127), axis=0,
                    keepdims=True).astype(jnp.float32)

    CH = 512
    for c in range(M // CH):
        lo, hi = c * CH, (c + 1) * CH
        idxr = jax.lax.broadcasted_iota(jnp.int32, (1, CH), 1) + lo
        srch = s_r[:, lo:hi]
        ix = jnp.maximum(
            jnp.minimum(x2c, x2r[:, lo:hi]) - jnp.maximum(x1c, x1r[:, lo:hi]), 0.0)
        iy = jnp.maximum(
            jnp.minimum(y2c, y2r[:, lo:hi]) - jnp.maximum(y1c, y1r[:, lo:hi]), 0.0)
        inter = ix * iy
        den = jnp.maximum(jnp.minimum(areac, arear[:, lo:hi]), 1e-9)
        over = inter / den > _NMS_T
        prior = (s_c > srch) | ((s_c == srch) & (idxc < idxr))
        mf_ref[:, lo:hi] = jnp.where(over & prior, 1.0, 0.0).astype(jnp.bfloat16)

    validf = jnp.where(s_r > 0.0, 1.0, 0.0).astype(jnp.bfloat16)

    def cond(carry):
        return carry[1]

    def body(carry):
        keep, _ = carry
        sup = jnp.dot(keep, mf_ref[...], preferred_element_type=jnp.float32) > 0.0
        new = jnp.where(sup, jnp.bfloat16(0.0), validf)
        d = (new - keep).astype(jnp.float32)
        return new, jnp.sum(d * d) > 0.0

    keep, _ = jax.lax.while_loop(cond, body, (validf, jnp.bool_(True)))

    asm_ref[...] = bt_ref[0:16, :]
    asm_ref[6:7, :] = cls_r
    bk = asm_ref[...] * keep.astype(jnp.float32)

    rank = jnp.zeros((M, 1), jnp.int32)
    for c in range(M // CH):
        lo, hi = c * CH, (c + 1) * CH
        idxr = jax.lax.broadcasted_iota(jnp.int32, (1, CH), 1) + lo
        srch = s_r[:, lo:hi]
        cmp = (srch > s_c) | ((srch == s_c) & (idxr < idxc))
        rank = rank + jnp.sum(jnp.where(cmp, 1, 0), axis=1, keepdims=True)

    for c in range(M // CH):
        lo, hi = c * CH, (c + 1) * CH
        col = jax.lax.broadcasted_iota(jnp.int32, (1, CH), 1) + lo
        pt = jnp.where(rank == col, 1.0, 0.0)
        out_ref[:, lo:hi] = jnp.dot(bk, pt, preferred_element_type=jnp.float32,
                                    precision=jax.lax.Precision.HIGHEST)


def _nms(bt):
    out_t = pl.pallas_call(
        _nms_body,
        in_specs=[
            pl.BlockSpec((_M, _NR), lambda: (0, 0)),
            pl.BlockSpec((_NR, _M), lambda: (0, 0)),
        ],
        out_specs=pl.BlockSpec((_NF, _M), lambda: (0, 0)),
        out_shape=jax.ShapeDtypeStruct((_NF, _M), jnp.float32),
        scratch_shapes=[pltpu.VMEM((_M, _M), jnp.bfloat16),
                        pltpu.VMEM((_NF, _M), jnp.float32)],
    )(bt.T, bt)
    return out_t[:9, :].T


def kernel(out13, out26, out52, anchors13, anchors26, anchors52):
    N = out13.shape[0]
    o4s = [o.reshape(N, 3, 85, o.shape[2] * o.shape[3])
           for o in (out13, out26, out52)]
    objlogits = _extract_obj(*o4s)
    cols = []
    for o4, logit, t, anch, W in zip(
            o4s, objlogits, (32.0, 16.0, 8.0),
            (anchors13, anchors26, anchors52),
            (out13.shape[3], out26.shape[3], out52.shape[3])):
        HW = o4.shape[3]
        obj = jax.nn.sigmoid(logit)
        score = jnp.where(obj > _THRESH, obj, _NEG)
        score = score.transpose(0, 2, 1).reshape(-1)
        top_s, top_i = jax.lax.top_k(score, _K)
        n = top_i // (3 * HW)
        rem = top_i % (3 * HW)
        s = rem // 3
        a = rem % 3
        raw = o4[n, a, :, s]
        gyf = (s // W).astype(jnp.float32)
        gxf = (s % W).astype(jnp.float32)
        cx = (gxf + raw[:, 1]) * t / _CASE
        cy = (gyf + raw[:, 2]) * t / _CASE
        w = jnp.take(anch[:, 0], a) * jnp.exp(raw[:, 3]) / _CASE
        h = jnp.take(anch[:, 1], a) * jnp.exp(raw[:, 4]) / _CASE
        obj_f = jnp.maximum(top_s, 0.0)
        zero = jnp.zeros_like(top_s)
        fields = jnp.stack([
            n.astype(jnp.float32), cx, cy, w, h, obj_f, zero, gyf, gxf,
            top_s, zero, zero, zero, zero, zero, zero,
        ])
        cols.append(jnp.concatenate([fields, raw[:, 5:85].T], axis=0))
    return _nms(jnp.concatenate(cols, axis=1))


# speedup vs baseline: 1.1451x; 1.1451x over previous
import jax
import jax.numpy as jnp
from jax.experimental import pallas as pl
from jax.experimental.pallas import tpu as pltpu

_THRESH = 0.6
_NMS_T = 0.7
_CASE = 416.0
_K = 512
_NEG = -1e9
_M = 3 * _K
_NR = 96
_NF = 16


def _thresh_body(o13_ref, o26_ref, o52_ref, s13_ref, s26_ref, s52_ref):
    for o_ref, s_ref in ((o13_ref, s13_ref), (o26_ref, s26_ref),
                         (o52_ref, s52_ref)):
        obj = o_ref[...]
        s_ref[...] = jnp.where(obj > _THRESH, obj, _NEG)


def _thresh_filter(obj13, obj26, obj52):
    return pl.pallas_call(
        _thresh_body,
        in_specs=[pl.BlockSpec(o.shape, lambda: (0, 0))
                  for o in (obj13, obj26, obj52)],
        out_specs=[pl.BlockSpec(o.shape, lambda: (0, 0))
                   for o in (obj13, obj26, obj52)],
        out_shape=[jax.ShapeDtypeStruct(o.shape, jnp.float32)
                   for o in (obj13, obj26, obj52)],
    )(obj13, obj26, obj52)


def _nms_body(b_ref, bt_ref, out_ref, mf_ref, asm_ref):
    M = _M
    s_c = b_ref[:, 9:10]
    x1c = b_ref[:, 1:2] - 0.5 * b_ref[:, 3:4]
    x2c = b_ref[:, 1:2] + 0.5 * b_ref[:, 3:4]
    y1c = b_ref[:, 2:3] - 0.5 * b_ref[:, 4:5]
    y2c = b_ref[:, 2:3] + 0.5 * b_ref[:, 4:5]
    areac = jnp.maximum(x2c - x1c, 0.0) * jnp.maximum(y2c - y1c, 0.0)
    idxc = jax.lax.broadcasted_iota(jnp.int32, (M, 1), 0)
    s_r = bt_ref[9:10, :]
    x1r = bt_ref[1:2, :] - 0.5 * bt_ref[3:4, :]
    x2r = bt_ref[1:2, :] + 0.5 * bt_ref[3:4, :]
    y1r = bt_ref[2:3, :] - 0.5 * bt_ref[4:5, :]
    y2r = bt_ref[2:3, :] + 0.5 * bt_ref[4:5, :]
    arear = jnp.maximum(x2r - x1r, 0.0) * jnp.maximum(y2r - y1r, 0.0)

    cl = bt_ref[16:96, :]
    mx = jnp.max(cl, axis=0, keepdims=True)
    ii = jax.lax.broadcasted_iota(jnp.int32, (80, M), 0)
    cls_r = jnp.min(jnp.where(cl == mx, ii, 127), axis=0,
                    keepdims=True).astype(jnp.float32)

    CH = 512
    for c in range(M // CH):
        lo, hi = c * CH, (c + 1) * CH
        idxr = jax.lax.broadcasted_iota(jnp.int32, (1, CH), 1) + lo
        srch = s_r[:, lo:hi]
        ix = jnp.maximum(
            jnp.minimum(x2c, x2r[:, lo:hi]) - jnp.maximum(x1c, x1r[:, lo:hi]), 0.0)
        iy = jnp.maximum(
            jnp.minimum(y2c, y2r[:, lo:hi]) - jnp.maximum(y1c, y1r[:, lo:hi]), 0.0)
        inter = ix * iy
        den = jnp.maximum(jnp.minimum(areac, arear[:, lo:hi]), 1e-9)
        over = inter / den > _NMS_T
        prior = (s_c > srch) | ((s_c == srch) & (idxc < idxr))
        mf_ref[:, lo:hi] = jnp.where(over & prior, 1.0, 0.0).astype(jnp.bfloat16)

    validf = jnp.where(s_r > 0.0, 1.0, 0.0).astype(jnp.bfloat16)

    def cond(carry):
        return carry[1]

    def body(carry):
        keep, _ = carry
        sup = jnp.dot(keep, mf_ref[...], preferred_element_type=jnp.float32) > 0.0
        new = jnp.where(sup, jnp.bfloat16(0.0), validf)
        d = (new - keep).astype(jnp.float32)
        return new, jnp.sum(d * d) > 0.0

    keep, _ = jax.lax.while_loop(cond, body, (validf, jnp.bool_(True)))

    asm_ref[...] = bt_ref[0:16, :]
    asm_ref[6:7, :] = cls_r
    bk = asm_ref[...] * keep.astype(jnp.float32)

    rank = jnp.zeros((M, 1), jnp.int32)
    for c in range(M // CH):
        lo, hi = c * CH, (c + 1) * CH
        idxr = jax.lax.broadcasted_iota(jnp.int32, (1, CH), 1) + lo
        srch = s_r[:, lo:hi]
        cmp = (srch > s_c) | ((srch == s_c) & (idxr < idxc))
        rank = rank + jnp.sum(jnp.where(cmp, 1, 0), axis=1, keepdims=True)

    for c in range(M // CH):
        lo, hi = c * CH, (c + 1) * CH
        col = jax.lax.broadcasted_iota(jnp.int32, (1, CH), 1) + lo
        pt = jnp.where(rank == col, 1.0, 0.0)
        out_ref[:, lo:hi] = jnp.dot(bk, pt, preferred_element_type=jnp.float32,
                                    precision=jax.lax.Precision.HIGHEST)


def _nms(bt):
    out_t = pl.pallas_call(
        _nms_body,
        in_specs=[
            pl.BlockSpec((_M, _NR), lambda: (0, 0)),
            pl.BlockSpec((_NR, _M), lambda: (0, 0)),
        ],
        out_specs=pl.BlockSpec((_NF, _M), lambda: (0, 0)),
        out_shape=jax.ShapeDtypeStruct((_NF, _M), jnp.float32),
        scratch_shapes=[pltpu.VMEM((_M, _M), jnp.bfloat16),
                        pltpu.VMEM((_NF, _M), jnp.float32)],
    )(bt.T, bt)
    return out_t[:9, :].T


def kernel(out13, out26, out52, anchors13, anchors26, anchors52):
    N = out13.shape[0]
    o4s = [o.reshape(N, 3, 85, o.shape[2] * o.shape[3])
           for o in (out13, out26, out52)]
    objs = [jax.nn.sigmoid(o4[:, :, 0, :]).transpose(0, 2, 1)
            .reshape(N, -1) for o4 in o4s]
    scores = _thresh_filter(*objs)
    cols = []
    for o4, score, t, anch, W in zip(
            o4s, scores, (32.0, 16.0, 8.0),
            (anchors13, anchors26, anchors52),
            (out13.shape[3], out26.shape[3], out52.shape[3])):
        HW = o4.shape[3]
        top_s, top_i = jax.lax.top_k(score.reshape(-1), _K)
        n = top_i // (3 * HW)
        rem = top_i % (3 * HW)
        s = rem // 3
        a = rem % 3
        raw = o4[n, a, :, s]
        gyf = (s // W).astype(jnp.float32)
        gxf = (s % W).astype(jnp.float32)
        cx = (gxf + raw[:, 1]) * t / _CASE
        cy = (gyf + raw[:, 2]) * t / _CASE
        w = jnp.take(anch[:, 0], a) * jnp.exp(raw[:, 3]) / _CASE
        h = jnp.take(anch[:, 1], a) * jnp.exp(raw[:, 4]) / _CASE
        obj_f = jnp.maximum(top_s, 0.0)
        zero = jnp.zeros_like(top_s)
        fields = jnp.stack([
            n.astype(jnp.float32), cx, cy, w, h, obj_f, zero, gyf, gxf,
            top_s, zero, zero, zero, zero, zero, zero,
        ])
        cols.append(jnp.concatenate([fields, raw[:, 5:85].T], axis=0))
    return _nms(jnp.concatenate(cols, axis=1))


# speedup vs baseline: 1.2394x; 1.0823x over previous
import jax
import jax.numpy as jnp
from jax.experimental import pallas as pl
from jax.experimental.pallas import tpu as pltpu

_THRESH = 0.6
_NMS_T = 0.7
_CASE = 416.0
_K = 512
_NEG = -1e9
_M = 3 * _K
_NR = 96
_NF = 16


def _thresh_body(o13_ref, o26_ref, o52_ref, s13_ref, s26_ref, s52_ref):
    for o_ref, s_ref in ((o13_ref, s13_ref), (o26_ref, s26_ref),
                         (o52_ref, s52_ref)):
        obj = o_ref[...]
        s_ref[...] = jnp.where(obj > _THRESH, obj, _NEG)


def _thresh_filter(obj13, obj26, obj52):
    return pl.pallas_call(
        _thresh_body,
        in_specs=[pl.BlockSpec(o.shape, lambda: (0, 0))
                  for o in (obj13, obj26, obj52)],
        out_specs=[pl.BlockSpec(o.shape, lambda: (0, 0))
                   for o in (obj13, obj26, obj52)],
        out_shape=[jax.ShapeDtypeStruct(o.shape, jnp.float32)
                   for o in (obj13, obj26, obj52)],
    )(obj13, obj26, obj52)


def _nms_body(b_ref, bt_ref, out_ref, mf_ref, asm_ref):
    M = _M
    s_c = b_ref[:, 9:10]
    x1c = b_ref[:, 1:2] - 0.5 * b_ref[:, 3:4]
    x2c = b_ref[:, 1:2] + 0.5 * b_ref[:, 3:4]
    y1c = b_ref[:, 2:3] - 0.5 * b_ref[:, 4:5]
    y2c = b_ref[:, 2:3] + 0.5 * b_ref[:, 4:5]
    areac = jnp.maximum(x2c - x1c, 0.0) * jnp.maximum(y2c - y1c, 0.0)
    idxc = jax.lax.broadcasted_iota(jnp.int32, (M, 1), 0)
    s_r = bt_ref[9:10, :]
    x1r = bt_ref[1:2, :] - 0.5 * bt_ref[3:4, :]
    x2r = bt_ref[1:2, :] + 0.5 * bt_ref[3:4, :]
    y1r = bt_ref[2:3, :] - 0.5 * bt_ref[4:5, :]
    y2r = bt_ref[2:3, :] + 0.5 * bt_ref[4:5, :]
    arear = jnp.maximum(x2r - x1r, 0.0) * jnp.maximum(y2r - y1r, 0.0)

    cl = bt_ref[16:96, :]
    mx = jnp.max(cl, axis=0, keepdims=True)
    ii = jax.lax.broadcasted_iota(jnp.int32, (80, M), 0)
    cls_r = jnp.min(jnp.where(cl == mx, ii, 127), axis=0,
                    keepdims=True).astype(jnp.float32)

    CH = 512
    for c in range(M // CH):
        lo, hi = c * CH, (c + 1) * CH
        idxr = jax.lax.broadcasted_iota(jnp.int32, (1, CH), 1) + lo
        srch = s_r[:, lo:hi]
        ix = jnp.maximum(
            jnp.minimum(x2c, x2r[:, lo:hi]) - jnp.maximum(x1c, x1r[:, lo:hi]), 0.0)
        iy = jnp.maximum(
            jnp.minimum(y2c, y2r[:, lo:hi]) - jnp.maximum(y1c, y1r[:, lo:hi]), 0.0)
        inter = ix * iy
        den = jnp.maximum(jnp.minimum(areac, arear[:, lo:hi]), 1e-9)
        over = inter / den > _NMS_T
        prior = (s_c > srch) | ((s_c == srch) & (idxc < idxr))
        mf_ref[:, lo:hi] = jnp.where(over & prior, 1.0, 0.0).astype(jnp.bfloat16)

    validf = jnp.where(s_r > 0.0, 1.0, 0.0).astype(jnp.bfloat16)

    def cond(carry):
        return carry[1]

    def body(carry):
        keep, _ = carry
        sup = jnp.dot(keep, mf_ref[...], preferred_element_type=jnp.float32) > 0.0
        new = jnp.where(sup, jnp.bfloat16(0.0), validf)
        d = (new - keep).astype(jnp.float32)
        return new, jnp.sum(d * d) > 0.0

    keep, _ = jax.lax.while_loop(cond, body, (validf, jnp.bool_(True)))

    asm_ref[...] = bt_ref[0:16, :]
    asm_ref[6:7, :] = cls_r
    bk = asm_ref[...] * keep.astype(jnp.float32)

    rank = jnp.zeros((M, 1), jnp.int32)
    for c in range(M // CH):
        lo, hi = c * CH, (c + 1) * CH
        idxr = jax.lax.broadcasted_iota(jnp.int32, (1, CH), 1) + lo
        srch = s_r[:, lo:hi]
        cmp = (srch > s_c) | ((srch == s_c) & (idxr < idxc))
        rank = rank + jnp.sum(jnp.where(cmp, 1, 0), axis=1, keepdims=True)

    for c in range(M // CH):
        lo, hi = c * CH, (c + 1) * CH
        col = jax.lax.broadcasted_iota(jnp.int32, (1, CH), 1) + lo
        pt = jnp.where(rank == col, 1.0, 0.0)
        out_ref[:, lo:hi] = jnp.dot(bk, pt, preferred_element_type=jnp.float32,
                                    precision=jax.lax.Precision.HIGHEST)


def _nms(bt):
    out_t = pl.pallas_call(
        _nms_body,
        in_specs=[
            pl.BlockSpec((_M, _NR), lambda: (0, 0)),
            pl.BlockSpec((_NR, _M), lambda: (0, 0)),
        ],
        out_specs=pl.BlockSpec((_NF, _M), lambda: (0, 0)),
        out_shape=jax.ShapeDtypeStruct((_NF, _M), jnp.float32),
        scratch_shapes=[pltpu.VMEM((_M, _M), jnp.bfloat16),
                        pltpu.VMEM((_NF, _M), jnp.float32)],
    )(bt.T, bt)
    return out_t[:9, :].T


def kernel(out13, out26, out52, anchors13, anchors26, anchors52):
    N = out13.shape[0]
    objs = [jax.nn.sigmoid(o[:, 0::85, :, :]).transpose(0, 2, 3, 1)
            .reshape(N, -1) for o in (out13, out26, out52)]
    scores = _thresh_filter(*objs)
    cols = []
    for o, score, t, anch in zip(
            (out13, out26, out52), scores, (32.0, 16.0, 8.0),
            (anchors13, anchors26, anchors52)):
        W = o.shape[3]
        HW = o.shape[2] * W
        top_s, top_i = jax.lax.top_k(score.reshape(-1), _K)
        n = top_i // (3 * HW)
        rem = top_i % (3 * HW)
        s = rem // 3
        a = rem % 3
        h = s // W
        ww = s % W
        ch = a[:, None] * 85 + jnp.arange(85, dtype=top_i.dtype)[None, :]
        raw = jnp.take_along_axis(o[n, :, h, ww], ch, axis=1)
        gyf = h.astype(jnp.float32)
        gxf = ww.astype(jnp.float32)
        cx = (gxf + raw[:, 1]) * t / _CASE
        cy = (gyf + raw[:, 2]) * t / _CASE
        w = jnp.take(anch[:, 0], a) * jnp.exp(raw[:, 3]) / _CASE
        h = jnp.take(anch[:, 1], a) * jnp.exp(raw[:, 4]) / _CASE
        obj_f = jnp.maximum(top_s, 0.0)
        zero = jnp.zeros_like(top_s)
        fields = jnp.stack([
            n.astype(jnp.float32), cx, cy, w, h, obj_f, zero, gyf, gxf,
            top_s, zero, zero, zero, zero, zero, zero,
        ])
        cols.append(jnp.concatenate([fields, raw[:, 5:85].T], axis=0))
    return _nms(jnp.concatenate(cols, axis=1))


# speedup vs baseline: 1.4568x; 1.1754x over previous
import jax
import jax.numpy as jnp
from jax.experimental import pallas as pl
from jax.experimental.pallas import tpu as pltpu

_THRESH = 0.6
_NMS_T = 0.7
_CASE = 416.0
_K = 512
_NEG = -1e9
_M = 3 * _K
_NR = 96
_NF = 16


def _thresh_body(o13_ref, o26_ref, o52_ref, s13_ref, s26_ref, s52_ref):
    for o_ref, s_ref in ((o13_ref, s13_ref), (o26_ref, s26_ref),
                         (o52_ref, s52_ref)):
        obj = o_ref[...]
        s_ref[...] = jnp.where(obj > _THRESH, obj, _NEG)


def _thresh_filter(obj13, obj26, obj52):
    return pl.pallas_call(
        _thresh_body,
        in_specs=[pl.BlockSpec(o.shape, lambda: (0, 0))
                  for o in (obj13, obj26, obj52)],
        out_specs=[pl.BlockSpec(o.shape, lambda: (0, 0))
                   for o in (obj13, obj26, obj52)],
        out_shape=[jax.ShapeDtypeStruct(o.shape, jnp.float32)
                   for o in (obj13, obj26, obj52)],
    )(obj13, obj26, obj52)


def _nms_body(b_ref, bt_ref, out_ref, mf_ref, asm_ref):
    M = _M
    s_c = b_ref[:, 9:10]
    x1c = b_ref[:, 1:2] - 0.5 * b_ref[:, 3:4]
    x2c = b_ref[:, 1:2] + 0.5 * b_ref[:, 3:4]
    y1c = b_ref[:, 2:3] - 0.5 * b_ref[:, 4:5]
    y2c = b_ref[:, 2:3] + 0.5 * b_ref[:, 4:5]
    areac = jnp.maximum(x2c - x1c, 0.0) * jnp.maximum(y2c - y1c, 0.0)
    idxc = jax.lax.broadcasted_iota(jnp.int32, (M, 1), 0)
    s_r = bt_ref[9:10, :]
    x1r = bt_ref[1:2, :] - 0.5 * bt_ref[3:4, :]
    x2r = bt_ref[1:2, :] + 0.5 * bt_ref[3:4, :]
    y1r = bt_ref[2:3, :] - 0.5 * bt_ref[4:5, :]
    y2r = bt_ref[2:3, :] + 0.5 * bt_ref[4:5, :]
    arear = jnp.maximum(x2r - x1r, 0.0) * jnp.maximum(y2r - y1r, 0.0)

    cl = bt_ref[16:96, :]
    mx = jnp.max(cl, axis=0, keepdims=True)
    ii = jax.lax.broadcasted_iota(jnp.int32, (80, M), 0)
    cls_r = jnp.min(jnp.where(cl == mx, ii, 127), axis=0,
                    keepdims=True).astype(jnp.float32)

    CH = 512
    for c in range(M // CH):
        lo, hi = c * CH, (c + 1) * CH
        idxr = jax.lax.broadcasted_iota(jnp.int32, (1, CH), 1) + lo
        srch = s_r[:, lo:hi]
        ix = jnp.maximum(
            jnp.minimum(x2c, x2r[:, lo:hi]) - jnp.maximum(x1c, x1r[:, lo:hi]), 0.0)
        iy = jnp.maximum(
            jnp.minimum(y2c, y2r[:, lo:hi]) - jnp.maximum(y1c, y1r[:, lo:hi]), 0.0)
        inter = ix * iy
        den = jnp.maximum(jnp.minimum(areac, arear[:, lo:hi]), 1e-9)
        over = inter / den > _NMS_T
        prior = (s_c > srch) | ((s_c == srch) & (idxc < idxr))
        mf_ref[:, lo:hi] = jnp.where(over & prior, 1.0, 0.0).astype(jnp.bfloat16)

    validf = jnp.where(s_r > 0.0, 1.0, 0.0).astype(jnp.bfloat16)

    def cond(carry):
        return carry[1]

    def body(carry):
        keep, _ = carry
        sup = jnp.dot(keep, mf_ref[...], preferred_element_type=jnp.float32) > 0.0
        new = jnp.where(sup, jnp.bfloat16(0.0), validf)
        d = (new - keep).astype(jnp.float32)
        return new, jnp.sum(d * d) > 0.0

    keep, _ = jax.lax.while_loop(cond, body, (validf, jnp.bool_(True)))

    asm_ref[...] = bt_ref[0:16, :]
    asm_ref[6:7, :] = cls_r
    bk = asm_ref[...] * keep.astype(jnp.float32)

    rank = jnp.zeros((M, 1), jnp.int32)
    for c in range(M // CH):
        lo, hi = c * CH, (c + 1) * CH
        idxr = jax.lax.broadcasted_iota(jnp.int32, (1, CH), 1) + lo
        srch = s_r[:, lo:hi]
        cmp = (srch > s_c) | ((srch == s_c) & (idxr < idxc))
        rank = rank + jnp.sum(jnp.where(cmp, 1, 0), axis=1, keepdims=True)

    for c in range(M // CH):
        lo, hi = c * CH, (c + 1) * CH
        col = jax.lax.broadcasted_iota(jnp.int32, (1, CH), 1) + lo
        pt = jnp.where(rank == col, 1.0, 0.0)
        out_ref[:, lo:hi] = jnp.dot(bk, pt, preferred_element_type=jnp.float32,
                                    precision=jax.lax.Precision.HIGHEST)


def _nms(bt):
    out_t = pl.pallas_call(
        _nms_body,
        in_specs=[
            pl.BlockSpec((_M, _NR), lambda: (0, 0)),
            pl.BlockSpec((_NR, _M), lambda: (0, 0)),
        ],
        out_specs=pl.BlockSpec((_NF, _M), lambda: (0, 0)),
        out_shape=jax.ShapeDtypeStruct((_NF, _M), jnp.float32),
        scratch_shapes=[pltpu.VMEM((_M, _M), jnp.bfloat16),
                        pltpu.VMEM((_NF, _M), jnp.float32)],
    )(bt.T, bt)
    return out_t[:9, :].T


def kernel(out13, out26, out52, anchors13, anchors26, anchors52):
    N = out13.shape[0]
    objs = [jax.nn.sigmoid(jnp.concatenate(
                [o[:, c:c + 1, :, :] for c in (0, 85, 170)], axis=1))
            .transpose(0, 2, 3, 1).reshape(N, -1)
            for o in (out13, out26, out52)]
    scores = _thresh_filter(*objs)
    cols = []
    for o, score, t, anch in zip(
            (out13, out26, out52), scores, (32.0, 16.0, 8.0),
            (anchors13, anchors26, anchors52)):
        W = o.shape[3]
        HW = o.shape[2] * W
        top_s, top_i = jax.lax.top_k(score.reshape(-1), _K)
        n = top_i // (3 * HW)
        rem = top_i % (3 * HW)
        s = rem // 3
        a = rem % 3
        h = s // W
        ww = s % W
        ch = a[:, None] * 85 + jnp.arange(85, dtype=top_i.dtype)[None, :]
        raw = jnp.take_along_axis(o[n, :, h, ww], ch, axis=1)
        gyf = h.astype(jnp.float32)
        gxf = ww.astype(jnp.float32)
        cx = (gxf + raw[:, 1]) * t / _CASE
        cy = (gyf + raw[:, 2]) * t / _CASE
        w = jnp.take(anch[:, 0], a) * jnp.exp(raw[:, 3]) / _CASE
        h = jnp.take(anch[:, 1], a) * jnp.exp(raw[:, 4]) / _CASE
        obj_f = jnp.maximum(top_s, 0.0)
        zero = jnp.zeros_like(top_s)
        fields = jnp.stack([
            n.astype(jnp.float32), cx, cy, w, h, obj_f, zero, gyf, gxf,
            top_s, zero, zero, zero, zero, zero, zero,
        ])
        cols.append(jnp.concatenate([fields, raw[:, 5:85].T], axis=0))
    return _nms(jnp.concatenate(cols, axis=1))
